# baseline (speedup 1.0000x reference)
import functools

import jax
import jax.numpy as jnp
from jax import lax
from jax.experimental import pallas as pl
from jax.experimental.pallas import tpu as pltpu
from jax.experimental.pallas import tpu_sc as plsc

BINS = 64
OUT_DIM = BINS * BINS
BATCH = 8
SAMPLES = 2048
NUM_CORES = 2
NUM_SUBCORES = 16
LANES = 16

B_PER_CORE = BATCH // NUM_CORES
TILES_PER_BATCH = NUM_SUBCORES // B_PER_CORE
S_PER_TILE = SAMPLES // TILES_PER_BATCH
HIST = B_PER_CORE * OUT_DIM
HIST_SLICE = HIST // NUM_SUBCORES
CHUNK = 128
N_CHUNKS = S_PER_TILE // CHUNK
WEIGHT = 1.0 / SAMPLES


def _bucket(x):
    t = x * 32.0
    t = jnp.minimum(jnp.maximum(t, -33.0), 33.0)
    ti = t.astype(jnp.int32)
    tf = ti.astype(jnp.float32)
    ceil_t = jnp.where(tf < t, ti + 1, ti)
    return jnp.minimum(jnp.maximum(ceil_t + 31, 0), BINS - 1)


@functools.partial(
    pl.kernel,
    out_type=jax.ShapeDtypeStruct((BATCH, OUT_DIM), jnp.float32),
    mesh=plsc.VectorSubcoreMesh(core_axis_name="c", subcore_axis_name="s"),
    scratch_types=[
        pltpu.VMEM((S_PER_TILE * 2,), jnp.float32),
        pltpu.VMEM((N_CHUNKS, CHUNK), jnp.int32),
        pltpu.VMEM((CHUNK,), jnp.float32),
        pltpu.VMEM((HIST_SLICE,), jnp.float32),
        pltpu.VMEM_SHARED((HIST,), jnp.float32),
        pltpu.SemaphoreType.DMA,
    ],
)
def _bin_density_sc(states_hbm, out_hbm, xy_v, idx_v, val_v, bounce_v,
                    hist_sh, sem):
    c = lax.axis_index("c")
    s = lax.axis_index("s")
    batch = c * B_PER_CORE + s // TILES_PER_BATCH
    sample0 = (s % TILES_PER_BATCH) * S_PER_TILE

    in_cp = pltpu.async_copy(
        states_hbm.at[batch, pl.ds(sample0 * 2, S_PER_TILE * 2)], xy_v, sem)

    zeros16 = jnp.zeros((LANES,), jnp.float32)
    def _zero(i, carry):
        bounce_v[pl.ds(i * LANES, LANES)] = zeros16
        return carry
    lax.fori_loop(0, HIST_SLICE // LANES, _zero, 0)
    pltpu.sync_copy(bounce_v, hist_sh.at[pl.ds(s * HIST_SLICE, HIST_SLICE)])

    w16 = jnp.full((LANES,), WEIGHT, jnp.float32)
    def _fill(i, carry):
        val_v[pl.ds(i * LANES, LANES)] = w16
        return carry
    lax.fori_loop(0, CHUNK // LANES, _fill, 0)

    in_cp.wait()

    even1d = (lax.iota(jnp.int32, LANES) & 7) * 2
    even = lax.broadcast_in_dim(even1d, (LANES, 1), (0,))
    odd = lax.broadcast_in_dim(even1d + 1, (LANES, 1), (0,))
    dn = lax.GatherDimensionNumbers(
        offset_dims=(), collapsed_slice_dims=(0,), start_index_map=(0,))
    in_bounds = lax.GatherScatterMode.PROMISE_IN_BOUNDS
    low_half = lax.iota(jnp.int32, LANES) < 8

    def _deinterleave(v0, v1, sel):
        a = lax.gather(v0, sel, dn, (1,), mode=in_bounds)
        b = lax.gather(v1, sel, dn, (1,), mode=in_bounds)
        return jnp.where(low_half, a, b)

    hist_base = (s // TILES_PER_BATCH) * OUT_DIM
    for j in range(N_CHUNKS):
        def _index(i, carry):
            base = 2 * (j * CHUNK + i * LANES)
            v0 = xy_v[pl.ds(base, LANES)]
            v1 = xy_v[pl.ds(base + LANES, LANES)]
            x = _deinterleave(v0, v1, even)
            y = _deinterleave(v0, v1, odd)
            lin = hist_base + _bucket(y) * BINS + _bucket(x)
            idx_v[j, pl.ds(i * LANES, LANES)] = lin
            return carry
        lax.fori_loop(0, CHUNK // LANES, _index, 0)

    plsc.subcore_barrier()

    for j in range(N_CHUNKS):
        pltpu.sync_copy(val_v, hist_sh.at[idx_v.at[j]], add=True)

    plsc.subcore_barrier()

    pltpu.sync_copy(hist_sh.at[pl.ds(s * HIST_SLICE, HIST_SLICE)], bounce_v)
    out_row = c * B_PER_CORE + s // TILES_PER_BATCH
    out_col = (s % TILES_PER_BATCH) * HIST_SLICE
    pltpu.sync_copy(bounce_v, out_hbm.at[out_row, pl.ds(out_col, HIST_SLICE)])


def kernel(states):
    return _bin_density_sc(states.reshape(BATCH, SAMPLES * 2))


# speedup vs baseline: 1.0655x; 1.0655x over previous
import functools

import jax
import jax.numpy as jnp
from jax import lax
from jax.experimental import pallas as pl
from jax.experimental.pallas import tpu as pltpu
from jax.experimental.pallas import tpu_sc as plsc

BINS = 64
OUT_DIM = BINS * BINS
BATCH = 8
SAMPLES = 2048
NUM_CORES = 2
NUM_SUBCORES = 16
LANES = 16

B_PER_CORE = BATCH // NUM_CORES
TILES_PER_BATCH = NUM_SUBCORES // B_PER_CORE
S_PER_TILE = SAMPLES // TILES_PER_BATCH
HIST = B_PER_CORE * OUT_DIM
HIST_SLICE = HIST // NUM_SUBCORES
CHUNK = 128
N_CHUNKS = S_PER_TILE // CHUNK
WEIGHT = 1.0 / SAMPLES


def _bucket(x):
    t = x * 32.0
    t = jnp.minimum(jnp.maximum(t, -33.0), 33.0)
    ti = t.astype(jnp.int32)
    tf = ti.astype(jnp.float32)
    ceil_t = jnp.where(tf < t, ti + 1, ti)
    return jnp.minimum(jnp.maximum(ceil_t + 31, 0), BINS - 1)


@functools.partial(
    pl.kernel,
    out_type=jax.ShapeDtypeStruct((BATCH, OUT_DIM), jnp.float32),
    mesh=plsc.VectorSubcoreMesh(core_axis_name="c", subcore_axis_name="s"),
    scratch_types=[
        pltpu.VMEM((S_PER_TILE,), jnp.float32),
        pltpu.VMEM((S_PER_TILE,), jnp.float32),
        pltpu.VMEM((N_CHUNKS, CHUNK), jnp.int32),
        pltpu.VMEM((CHUNK,), jnp.float32),
        pltpu.VMEM((HIST_SLICE,), jnp.float32),
        pltpu.VMEM_SHARED((HIST,), jnp.float32),
        pltpu.SemaphoreType.DMA,
        pltpu.SemaphoreType.DMA,
    ],
)
def _bin_density_sc(states_hbm, out_hbm, x_v, y_v, idx_v, val_v, bounce_v,
                    hist_sh, sem, scat_sem):
    c = lax.axis_index("c")
    s = lax.axis_index("s")
    batch = c * B_PER_CORE + s // TILES_PER_BATCH
    sample0 = (s % TILES_PER_BATCH) * S_PER_TILE

    x_cp = pltpu.async_copy(
        states_hbm.at[batch, 0, pl.ds(sample0, S_PER_TILE)], x_v, sem)
    y_cp = pltpu.async_copy(
        states_hbm.at[batch, 1, pl.ds(sample0, S_PER_TILE)], y_v, sem)

    zeros16 = jnp.zeros((LANES,), jnp.float32)
    def _zero(i, carry):
        bounce_v[pl.ds(i * LANES, LANES)] = zeros16
        return carry
    lax.fori_loop(0, HIST_SLICE // LANES, _zero, 0)
    pltpu.sync_copy(bounce_v, hist_sh.at[pl.ds(s * HIST_SLICE, HIST_SLICE)])

    w16 = jnp.full((LANES,), WEIGHT, jnp.float32)
    def _fill(i, carry):
        val_v[pl.ds(i * LANES, LANES)] = w16
        return carry
    lax.fori_loop(0, CHUNK // LANES, _fill, 0)

    plsc.subcore_barrier()

    x_cp.wait()
    y_cp.wait()

    hist_base = (s // TILES_PER_BATCH) * OUT_DIM
    scat_cps = []
    for j in range(N_CHUNKS):
        def _index(i, carry):
            base = j * CHUNK + i * LANES
            x = x_v[pl.ds(base, LANES)]
            y = y_v[pl.ds(base, LANES)]
            lin = hist_base + _bucket(y) * BINS + _bucket(x)
            idx_v[j, pl.ds(i * LANES, LANES)] = lin
            return carry
        lax.fori_loop(0, CHUNK // LANES, _index, 0)
        scat_cps.append(pltpu.async_copy(
            val_v, hist_sh.at[idx_v.at[j]], scat_sem, add=True))

    for cp in scat_cps:
        cp.wait()
    plsc.subcore_barrier()

    out_row = c * B_PER_CORE + s // TILES_PER_BATCH
    out_col = (s % TILES_PER_BATCH) * HIST_SLICE
    pltpu.sync_copy(hist_sh.at[pl.ds(s * HIST_SLICE, HIST_SLICE)],
                    out_hbm.at[out_row, pl.ds(out_col, HIST_SLICE)])


def kernel(states):
    return _bin_density_sc(states.transpose(0, 2, 1))
